# baseline (speedup 1.0000x reference)
import functools
import math

import jax
import jax.numpy as jnp
from jax import lax
from jax.experimental import pallas as pl
from jax.experimental.pallas import tpu as pltpu
from jax.experimental.pallas import tpu_sc as plsc

D_MODEL = 1024
SCALE = math.sqrt(D_MODEL)
L = 16
NC, NS = 2, 16
NW = NC * NS

CHUNK = 16
NBUF = 7
LEAD = 6


def _make_emb(SB: int, SEQ: int, D: int):
    B = SB * SEQ
    bpw = B // NW
    nchunk = bpw // CHUNK
    wpr = SEQ // bpw
    mesh = plsc.VectorSubcoreMesh(core_axis_name="c", subcore_axis_name="s")

    @functools.partial(
        pl.kernel,
        mesh=mesh,
        out_type=jax.ShapeDtypeStruct((SB, SEQ, D), jnp.float32),
        scratch_types=[
            pltpu.VMEM((bpw,), jnp.int32),
            *[pltpu.VMEM((CHUNK, D), jnp.float32) for _ in range(NBUF)],
            *[pltpu.SemaphoreType.DMA for _ in range(2 * NBUF)],
        ],
    )
    def emb(idx_hbm, table_hbm, out_hbm, idx_v, *rest):
        bufs = rest[:NBUF]
        gsem = rest[NBUF : 2 * NBUF]
        wsem = rest[2 * NBUF :]

        wid = lax.axis_index("s") * NC + lax.axis_index("c")
        row = wid // wpr
        col = (wid % wpr) * bpw
        pltpu.sync_copy(idx_hbm.at[row, pl.ds(col, bpw)], idx_v)

        def gather(c):
            s = c % NBUF
            return pltpu.async_copy(
                table_hbm.at[idx_v.at[pl.ds(c * CHUNK, CHUNK)]], bufs[s], gsem[s]
            )

        def scale(s):
            def body(r, carry):
                for j in range(D // L):
                    sl = pl.ds(j * L, L)
                    bufs[s][r, sl] = bufs[s][r, sl] * SCALE
                return carry

            lax.fori_loop(0, CHUNK, body, 0)

        gd = [None] * nchunk
        wd = [None] * nchunk
        w_waited = [False] * nchunk
        for c in range(min(LEAD, nchunk)):
            gd[c] = gather(c)
        for c in range(nchunk):
            s = c % NBUF
            nxt = c + LEAD
            if nxt < nchunk:
                prev = nxt - NBUF
                if prev >= 0:
                    wd[prev].wait()
                    w_waited[prev] = True
                gd[nxt] = gather(nxt)
            gd[c].wait()
            scale(s)
            wd[c] = pltpu.async_copy(
                bufs[s], out_hbm.at[row, pl.ds(col + c * CHUNK, CHUNK)], wsem[s]
            )
        for c in range(nchunk):
            if not w_waited[c]:
                wd[c].wait()

    return emb


def kernel(x, table):
    b, s = x.shape
    v, d = table.shape
    return _make_emb(b, s, d)(x, table)


# speedup vs baseline: 1.1452x; 1.1452x over previous
import functools
import math

import jax
import jax.numpy as jnp
from jax import lax
from jax.experimental import pallas as pl
from jax.experimental.pallas import tpu as pltpu
from jax.experimental.pallas import tpu_sc as plsc

D_MODEL = 1024
SCALE = math.sqrt(D_MODEL)
L = 16
NC, NS = 2, 16
NW = NC * NS

CHUNK = 16
NBUF = 7
LEAD = 5


def _make_emb(SB: int, SEQ: int, D: int):
    B = SB * SEQ
    bpw = B // NW
    nchunk = bpw // CHUNK
    wpr = SEQ // bpw
    mesh = plsc.VectorSubcoreMesh(core_axis_name="c", subcore_axis_name="s")

    @functools.partial(
        pl.kernel,
        mesh=mesh,
        out_type=jax.ShapeDtypeStruct((SB, SEQ, D), jnp.float32),
        scratch_types=[
            pltpu.VMEM((bpw,), jnp.int32),
            *[pltpu.VMEM((CHUNK, D), jnp.float32) for _ in range(NBUF)],
            *[pltpu.SemaphoreType.DMA for _ in range(2 * NBUF)],
        ],
    )
    def emb(idx_hbm, table_hbm, out_hbm, idx_v, *rest):
        bufs = rest[:NBUF]
        gsem = rest[NBUF : 2 * NBUF]
        wsem = rest[2 * NBUF :]

        wid = lax.axis_index("s") * NC + lax.axis_index("c")
        row = wid // wpr
        col = (wid % wpr) * bpw
        pltpu.sync_copy(idx_hbm.at[row, pl.ds(col, bpw)], idx_v)

        H = CHUNK // 2

        def gather(c):
            s = c % NBUF
            d0 = pltpu.async_copy(
                table_hbm.at[idx_v.at[pl.ds(c * CHUNK, H)]],
                bufs[s].at[pl.ds(0, H)],
                gsem[s],
            )
            d1 = pltpu.async_copy(
                table_hbm.at[idx_v.at[pl.ds(c * CHUNK + H, H)]],
                bufs[s].at[pl.ds(H, H)],
                gsem[s],
            )
            return (d0, d1)

        def scale(s):
            def body(r, carry):
                for j in range(D // L):
                    sl = pl.ds(j * L, L)
                    bufs[s][r, sl] = bufs[s][r, sl] * SCALE
                return carry

            lax.fori_loop(0, CHUNK, body, 0)

        gd = [None] * nchunk
        wd = [None] * nchunk
        w_waited = [False] * nchunk
        for c in range(min(LEAD, nchunk)):
            gd[c] = gather(c)
        for c in range(nchunk):
            s = c % NBUF
            nxt = c + LEAD
            if nxt < nchunk:
                prev = nxt - NBUF
                if prev >= 0:
                    wd[prev].wait()
                    w_waited[prev] = True
                gd[nxt] = gather(nxt)
            gd[c][0].wait()
            gd[c][1].wait()
            scale(s)
            wd[c] = pltpu.async_copy(
                bufs[s], out_hbm.at[row, pl.ds(col + c * CHUNK, CHUNK)], wsem[s]
            )
        for c in range(nchunk):
            if not w_waited[c]:
                wd[c].wait()

    return emb


def kernel(x, table):
    b, s = x.shape
    v, d = table.shape
    return _make_emb(b, s, d)(x, table)


# speedup vs baseline: 1.1538x; 1.0075x over previous
import functools
import math

import jax
import jax.numpy as jnp
from jax import lax
from jax.experimental import pallas as pl
from jax.experimental.pallas import tpu as pltpu
from jax.experimental.pallas import tpu_sc as plsc

D_MODEL = 1024
SCALE = math.sqrt(D_MODEL)
L = 16
NC, NS = 2, 16
NW = NC * NS

CHUNK = 16
NBUF = 7
LEAD = 5


def _make_emb(SB: int, SEQ: int, D: int):
    B = SB * SEQ
    bpw = B // NW
    nchunk = bpw // CHUNK
    wpr = SEQ // bpw
    mesh = plsc.VectorSubcoreMesh(core_axis_name="c", subcore_axis_name="s")

    @functools.partial(
        pl.kernel,
        mesh=mesh,
        out_type=jax.ShapeDtypeStruct((SB, SEQ, D), jnp.float32),
        scratch_types=[
            pltpu.VMEM((bpw,), jnp.int32),
            *[pltpu.VMEM((CHUNK, D), jnp.float32) for _ in range(NBUF)],
            *[pltpu.SemaphoreType.DMA for _ in range(2 * NBUF)],
        ],
    )
    def emb(idx_hbm, table_hbm, out_hbm, idx_v, *rest):
        bufs = rest[:NBUF]
        gsem = rest[NBUF : 2 * NBUF]
        wsem = rest[2 * NBUF :]

        wid = lax.axis_index("s") * NC + lax.axis_index("c")
        row = wid // wpr
        col = (wid % wpr) * bpw
        pltpu.sync_copy(idx_hbm.at[row, pl.ds(col, bpw)], idx_v)

        def gather(c):
            s = c % NBUF
            return pltpu.async_copy(
                table_hbm.at[idx_v.at[pl.ds(c * CHUNK, CHUNK)]], bufs[s], gsem[s]
            )

        def scale(s):
            def body(r, carry):
                for j in range(D // L):
                    sl = pl.ds(j * L, L)
                    bufs[s][r, sl] = bufs[s][r, sl] * SCALE
                return carry

            lax.fori_loop(0, CHUNK, body, 0)

        gd = [None] * nchunk
        wd = [None] * nchunk
        w_waited = [False] * nchunk
        for c in range(min(LEAD, nchunk)):
            gd[c] = gather(c)
        for c in range(nchunk):
            s = c % NBUF
            nxt = c + LEAD
            if nxt < nchunk:
                prev = nxt - NBUF
                if prev >= 0:
                    wd[prev].wait()
                    w_waited[prev] = True
                gd[nxt] = gather(nxt)
            gd[c].wait()
            scale(s)
            wd[c] = pltpu.async_copy(
                bufs[s], out_hbm.at[row, pl.ds(col + c * CHUNK, CHUNK)], wsem[s]
            )
        for c in range(nchunk):
            if not w_waited[c]:
                wd[c].wait()

    return emb


def kernel(x, table):
    b, s = x.shape
    v, d = table.shape
    return _make_emb(b, s, d)(x, table)
